# baseline (device time: 21100 ns/iter reference)
import jax
import jax.numpy as jnp
from jax import lax
from jax.experimental import pallas as pl
from jax.experimental.pallas import tpu as pltpu

N_DEV = 4
B_LOC = 2
HQ = 16
HQ_LOC = 4
SQ = 128
DH = 64
D_MODEL = 512
D_HID = HQ_LOC * DH


def kernel(x, Wq, K_ext, V_ext, Wo):
    my = lax.axis_index("i")
    k_loc = lax.dynamic_slice(K_ext, (B_LOC * my, 0, 0, 0), (B_LOC, SQ, HQ, DH))
    v_loc = lax.dynamic_slice(V_ext, (B_LOC * my, 0, 0, 0), (B_LOC, SQ, HQ, DH))
    K_t = jnp.transpose(k_loc, (2, 0, 1, 3)).astype(jnp.bfloat16)
    V_t = jnp.transpose(v_loc, (2, 0, 1, 3)).astype(jnp.bfloat16)

    def body(x_ref, wq_ref, k_t, v_t, wo_ref, out_ref,
             wq_comm, wo_comm, xb_ref, ctx_ref,
             q_send, o_send, q_recv, o_recv):
        me = lax.axis_index("i")
        left = (me + N_DEV - 1) % N_DEV
        right = (me + 1) % N_DEV

        barrier = pltpu.get_barrier_semaphore()
        for nbr in (left, right):
            pl.semaphore_signal(barrier, inc=1, device_id=(nbr,),
                                device_id_type=pl.DeviceIdType.MESH)
        pl.semaphore_wait(barrier, 2)

        wq_comm[0] = wq_ref[...].astype(jnp.bfloat16)
        wo_comm[0] = wo_ref[...].astype(jnp.bfloat16)
        xb_ref[...] = x_ref[...].reshape(B_LOC * SQ, D_MODEL).astype(jnp.bfloat16)

        s_qr = pltpu.make_async_remote_copy(
            src_ref=wq_comm.at[0], dst_ref=wq_comm.at[1],
            send_sem=q_send.at[0], recv_sem=q_recv.at[1],
            device_id=(right,), device_id_type=pl.DeviceIdType.MESH)
        s_or = pltpu.make_async_remote_copy(
            src_ref=wo_comm.at[0], dst_ref=wo_comm.at[1],
            send_sem=o_send.at[0], recv_sem=o_recv.at[1],
            device_id=(right,), device_id_type=pl.DeviceIdType.MESH)
        s_ql = pltpu.make_async_remote_copy(
            src_ref=wq_comm.at[0], dst_ref=wq_comm.at[2],
            send_sem=q_send.at[1], recv_sem=q_recv.at[2],
            device_id=(left,), device_id_type=pl.DeviceIdType.MESH)
        s_ol = pltpu.make_async_remote_copy(
            src_ref=wo_comm.at[0], dst_ref=wo_comm.at[2],
            send_sem=o_send.at[1], recv_sem=o_recv.at[2],
            device_id=(left,), device_id_type=pl.DeviceIdType.MESH)
        s_qr.start()
        s_or.start()
        s_ql.start()
        s_ol.start()

        slot_origin_off = (0, N_DEV - 1, 1, 2)

        B2 = B_LOC * SQ
        row = lax.broadcasted_iota(jnp.int32, (B2, B2), 0)
        col = lax.broadcasted_iota(jnp.int32, (B2, B2), 1)
        same_batch = (row // SQ) == (col // SQ)
        causal = ((col % SQ) // 64) <= ((row % SQ) // 64)
        bias = jnp.where(same_batch & causal, 0.0, -1e9).astype(jnp.float32)

        def group_contribution(slot):
            g = (me + slot_origin_off[slot]) % N_DEV
            qg = lax.dot(xb_ref[...], wq_comm[slot],
                         preferred_element_type=jnp.float32
                         ).astype(jnp.bfloat16)
            for h in range(HQ_LOC):
                hh = g * HQ_LOC + h
                q = qg[:, h * DH:(h + 1) * DH]
                k = k_t[hh].reshape(B2, DH)
                s = lax.dot_general(
                    q, k, (((1,), (1,)), ((), ())),
                    preferred_element_type=jnp.float32)
                s = s * 0.125 + bias
                m = jnp.max(s, axis=1, keepdims=True)
                w = jnp.exp(s - m)
                p = (w / jnp.sum(w, axis=1, keepdims=True)).astype(jnp.bfloat16)
                v = v_t[hh].reshape(B2, DH)
                ctx_ref[:, h * DH:(h + 1) * DH] = (
                    lax.dot(p, v, preferred_element_type=jnp.float32)
                    .astype(jnp.bfloat16))
            return lax.dot(ctx_ref[...], wo_comm[slot],
                           preferred_element_type=jnp.float32)

        def recv(buf, slot, sems):
            return pltpu.make_async_remote_copy(
                src_ref=buf.at[0], dst_ref=buf.at[slot],
                send_sem=q_send.at[0], recv_sem=sems.at[slot],
                device_id=(left,), device_id_type=pl.DeviceIdType.MESH)

        acc = group_contribution(0)

        recv(wq_comm, 1, q_recv).wait_recv()
        s_qf = pltpu.make_async_remote_copy(
            src_ref=wq_comm.at[1], dst_ref=wq_comm.at[3],
            send_sem=q_send.at[2], recv_sem=q_recv.at[3],
            device_id=(right,), device_id_type=pl.DeviceIdType.MESH)
        s_qf.start()
        recv(wo_comm, 1, o_recv).wait_recv()
        acc = acc + group_contribution(1)

        recv(wo_comm, 2, o_recv).wait_recv()
        s_of = pltpu.make_async_remote_copy(
            src_ref=wo_comm.at[2], dst_ref=wo_comm.at[3],
            send_sem=o_send.at[2], recv_sem=o_recv.at[3],
            device_id=(left,), device_id_type=pl.DeviceIdType.MESH)
        s_of.start()
        recv(wq_comm, 2, q_recv).wait_recv()
        acc = acc + group_contribution(2)

        recv(wq_comm, 3, q_recv).wait_recv()
        recv(wo_comm, 3, o_recv).wait_recv()
        acc = acc + group_contribution(3)

        out_ref[...] = acc.reshape(B_LOC, SQ, D_MODEL)

        for d in (s_qr, s_or, s_ql, s_ol, s_qf, s_of):
            d.wait_send()

    return pl.pallas_call(
        body,
        out_shape=jax.ShapeDtypeStruct((B_LOC, SQ, D_MODEL), jnp.float32),
        in_specs=[
            pl.BlockSpec(memory_space=pltpu.VMEM),
            pl.BlockSpec(memory_space=pltpu.VMEM),
            pl.BlockSpec(memory_space=pltpu.VMEM),
            pl.BlockSpec(memory_space=pltpu.VMEM),
            pl.BlockSpec(memory_space=pltpu.VMEM),
        ],
        out_specs=pl.BlockSpec(memory_space=pltpu.VMEM),
        scratch_shapes=[
            pltpu.VMEM((N_DEV, D_MODEL, D_HID), jnp.bfloat16),
            pltpu.VMEM((N_DEV, D_HID, D_MODEL), jnp.bfloat16),
            pltpu.VMEM((B_LOC * SQ, D_MODEL), jnp.bfloat16),
            pltpu.VMEM((B_LOC * SQ, D_HID), jnp.bfloat16),
            pltpu.SemaphoreType.DMA((3,)),
            pltpu.SemaphoreType.DMA((3,)),
            pltpu.SemaphoreType.DMA((N_DEV,)),
            pltpu.SemaphoreType.DMA((N_DEV,)),
        ],
        compiler_params=pltpu.CompilerParams(collective_id=0),
    )(x, Wq, K_t, V_t, Wo)
